# baseline (speedup 1.0000x reference)
import jax
import jax.numpy as jnp
from jax import lax
from jax.experimental import pallas as pl
from jax.experimental.pallas import tpu as pltpu
from jax.experimental.pallas import tpu_sc as plsc

SEQ = 8192
D = 2048
NIMG = 1024
SCALE = 12.0

NC = 2
NS = 16
NW = NC * NS
RPW = SEQ // NW
CHUNK = 16
NCHUNK = RPW // CHUNK
NBUF = 3
LANES = 16


def _sc_body(ids_hbm, img_hbm, vis_hbm, tab_hbm, out_hbm,
             idx_v, img_v, b0, b1, b2, g0, g1, g2, w0, w1, w2, osem):
    bufs = (b0, b1, b2)
    gsems = (g0, g1, g2)
    wsems = (w0, w1, w2)

    wid = lax.axis_index("s") * NC + lax.axis_index("c")
    base = wid * RPW

    pltpu.sync_copy(ids_hbm.at[pl.ds(base, RPW)], idx_v)
    pltpu.sync_copy(img_hbm, img_v.at[pl.ds(0, NIMG)])
    img_v[pl.ds(NIMG, LANES)] = jnp.full((LANES,), SEQ + 1, jnp.int32)

    def img_at(p):
        return img_v[pl.ds(p, LANES)][0]

    def count_lt(bound):
        lo = jnp.int32(0)
        s = NIMG
        while s >= 1:
            cand = lo + s
            probe = jnp.minimum(cand, NIMG) - 1
            take = jnp.logical_and(cand <= NIMG, img_at(probe) < bound)
            lo = jnp.where(take, cand, lo)
            s //= 2
        return lo

    def gather(c):
        b = c % NBUF
        return pltpu.async_copy(
            tab_hbm.at[idx_v.at[pl.ds(c * CHUNK, CHUNK)]], bufs[b], gsems[b]
        )

    def overwrite(lo, hi, n_acc):
        @pl.loop(lo, hi, init_carry=n_acc)
        def n_new(pp, cnt):
            is_last = img_at(pp + 1) != img_at(pp)

            @pl.when(is_last)
            def _():
                row = img_at(pp)
                pltpu.async_copy(vis_hbm.at[pp], out_hbm.at[row], osem)

            return cnt + jnp.where(is_last, 1, 0)

        return n_new

    bounds = [count_lt(base + t * CHUNK) for t in range(NCHUNK + 1)]
    n_ov = jnp.int32(0)
    gathers = [gather(0), gather(1)]
    writes = [None] * NCHUNK

    for c in range(NCHUNK):
        b = c % NBUF
        buf = bufs[b]
        cb = base + c * CHUNK

        gathers[c].wait()

        nvec = D // LANES

        @plsc.parallel_loop(0, CHUNK * nvec, unroll=8)
        def _(m):
            r = m // nvec
            sl = pl.ds((m % nvec) * LANES, LANES)
            buf[r, sl] = buf[r, sl] * SCALE

        writes[c] = pltpu.async_copy(buf, out_hbm.at[pl.ds(cb, CHUNK)], wsems[b])

        if c + 2 < NCHUNK:
            if c >= 1:
                writes[c - 1].wait()
                n_ov = overwrite(bounds[c - 1], bounds[c], n_ov)
            gathers.append(gather(c + 2))

    for c in range(NCHUNK - 3, NCHUNK):
        writes[c].wait()
        n_ov = overwrite(bounds[c], bounds[c + 1], n_ov)

    @pl.loop(0, n_ov)
    def _(_k):
        pltpu.make_async_copy(vis_hbm.at[0], out_hbm.at[0], osem).wait()


def kernel(input_ids, image_indices, vision_hidden_states, embed_table):
    mesh = plsc.VectorSubcoreMesh(core_axis_name="c", subcore_axis_name="s")
    f = pl.kernel(
        _sc_body,
        out_type=jax.ShapeDtypeStruct((SEQ, D), jnp.float32),
        mesh=mesh,
        scratch_types=[
            pltpu.VMEM((RPW,), jnp.int32),
            pltpu.VMEM((NIMG + LANES,), jnp.int32),
            pltpu.VMEM((CHUNK, D), jnp.float32),
            pltpu.VMEM((CHUNK, D), jnp.float32),
            pltpu.VMEM((CHUNK, D), jnp.float32),
            pltpu.SemaphoreType.DMA,
            pltpu.SemaphoreType.DMA,
            pltpu.SemaphoreType.DMA,
            pltpu.SemaphoreType.DMA,
            pltpu.SemaphoreType.DMA,
            pltpu.SemaphoreType.DMA,
            pltpu.SemaphoreType.DMA,
        ],
    )
    return f(input_ids, image_indices, vision_hidden_states, embed_table)


# speedup vs baseline: 3.3680x; 3.3680x over previous
import jax
import jax.numpy as jnp
from jax import lax
from jax.experimental import pallas as pl
from jax.experimental.pallas import tpu as pltpu
from jax.experimental.pallas import tpu_sc as plsc

SEQ = 8192
D = 2048
NIMG = 1024
SCALE = 12.0

NC = 2
NS = 16
NW = NC * NS
RPW = SEQ // NW
CHUNK = 16
NCHUNK = RPW // CHUNK
NBUF = 3
LANES = 16


def _sc_body(ids_hbm, img_hbm, vis_hbm, tab_hbm, out_hbm,
             idx_v, img_v, b0, b1, b2, g0, g1, g2, w0, w1, w2, osem):
    bufs = (b0, b1, b2)
    gsems = (g0, g1, g2)
    wsems = (w0, w1, w2)

    wid = lax.axis_index("s") * NC + lax.axis_index("c")
    base = wid * RPW

    c0 = pltpu.async_copy(ids_hbm.at[pl.ds(base, RPW)], idx_v, osem)
    c1 = pltpu.async_copy(img_hbm, img_v.at[pl.ds(0, NIMG)], osem)
    c0.wait()
    c1.wait()
    img_v[pl.ds(NIMG, LANES)] = jnp.full((LANES,), SEQ + 1, jnp.int32)

    def img_at(p):
        return img_v[pl.ds(p, LANES)][0]

    def count_lt(bound):
        lo = jnp.int32(0)
        s = NIMG
        while s >= 1:
            cand = lo + s
            probe = jnp.minimum(cand, NIMG) - 1
            take = jnp.logical_and(cand <= NIMG, img_at(probe) < bound)
            lo = jnp.where(take, cand, lo)
            s //= 2
        return lo

    def gather(c):
        b = c % NBUF
        return pltpu.async_copy(
            tab_hbm.at[idx_v.at[pl.ds(c * CHUNK, CHUNK)]], bufs[b], gsems[b]
        )

    p = count_lt(base)
    gathers = [gather(0), gather(1)]
    writes = [None] * NCHUNK

    for c in range(NCHUNK):
        b = c % NBUF
        buf = bufs[b]
        cb = base + c * CHUNK

        gathers[c].wait()

        nvec = D // LANES

        hi = count_lt(base + (c + 1) * CHUNK)

        @pl.loop(p, hi, init_carry=(jnp.int32(0), jnp.int32(0)))
        def ov_state(pp, carry):
            cnt, rmask = carry
            r = img_at(pp) - cb
            is_last = img_at(pp + 1) != img_at(pp)

            @pl.when(is_last)
            def _():
                pltpu.async_copy(vis_hbm.at[pp], buf.at[r], osem)

            return (cnt + jnp.where(is_last, 1, 0),
                    rmask | lax.shift_left(jnp.int32(1), r))

        n_issued, row_mask = ov_state

        def row_body(r, carry):
            skip = (lax.shift_right_logical(row_mask, r) & 1) == 1

            @pl.when(jnp.logical_not(skip))
            def _():
                @plsc.parallel_loop(0, nvec, unroll=16)
                def _(m):
                    sl = pl.ds(m * LANES, LANES)
                    buf[r, sl] = buf[r, sl] * SCALE

            return carry

        lax.fori_loop(0, CHUNK, row_body, jnp.int32(0))

        @pl.loop(0, n_issued)
        def _(_k):
            pltpu.make_async_copy(vis_hbm.at[0], buf.at[0], osem).wait()

        p = hi

        writes[c] = pltpu.async_copy(buf, out_hbm.at[pl.ds(cb, CHUNK)], wsems[b])

        if c + 2 < NCHUNK:
            if c >= 1:
                writes[c - 1].wait()
            gathers.append(gather(c + 2))

    writes[NCHUNK - 3].wait()
    writes[NCHUNK - 2].wait()
    writes[NCHUNK - 1].wait()


def kernel(input_ids, image_indices, vision_hidden_states, embed_table):
    mesh = plsc.VectorSubcoreMesh(core_axis_name="c", subcore_axis_name="s")
    f = pl.kernel(
        _sc_body,
        out_type=jax.ShapeDtypeStruct((SEQ, D), jnp.float32),
        mesh=mesh,
        scratch_types=[
            pltpu.VMEM((RPW,), jnp.int32),
            pltpu.VMEM((NIMG + LANES,), jnp.int32),
            pltpu.VMEM((CHUNK, D), jnp.float32),
            pltpu.VMEM((CHUNK, D), jnp.float32),
            pltpu.VMEM((CHUNK, D), jnp.float32),
            pltpu.SemaphoreType.DMA,
            pltpu.SemaphoreType.DMA,
            pltpu.SemaphoreType.DMA,
            pltpu.SemaphoreType.DMA,
            pltpu.SemaphoreType.DMA,
            pltpu.SemaphoreType.DMA,
            pltpu.SemaphoreType.DMA,
        ],
    )
    return f(input_ids, image_indices, vision_hidden_states, embed_table)


# speedup vs baseline: 3.4166x; 1.0144x over previous
import jax
import jax.numpy as jnp
from jax import lax
from jax.experimental import pallas as pl
from jax.experimental.pallas import tpu as pltpu
from jax.experimental.pallas import tpu_sc as plsc

SEQ = 8192
D = 2048
NIMG = 1024
SCALE = 12.0

NC = 2
NS = 16
NW = NC * NS
RPW = SEQ // NW
CHUNK = 16
NCHUNK = RPW // CHUNK
NBUF = 3
LANES = 16


def _sc_body(ids_hbm, img_hbm, vis_hbm, tab_hbm, out_hbm,
             idx_v, img_v, b0, b1, b2, g0, g1, g2, w0, w1, w2, osem):
    bufs = (b0, b1, b2)
    gsems = (g0, g1, g2)
    wsems = (w0, w1, w2)

    wid = lax.axis_index("s") * NC + lax.axis_index("c")
    base = wid * RPW

    c0 = pltpu.async_copy(ids_hbm.at[pl.ds(base, RPW)], idx_v, osem)
    c1 = pltpu.async_copy(img_hbm, img_v.at[pl.ds(0, NIMG)], osem)

    def img_at(p):
        return img_v[pl.ds(p, LANES)][0]

    def count_lt(bound):
        lo = jnp.int32(0)
        s = NIMG
        while s >= 1:
            cand = lo + s
            probe = jnp.minimum(cand, NIMG) - 1
            take = jnp.logical_and(cand <= NIMG, img_at(probe) < bound)
            lo = jnp.where(take, cand, lo)
            s //= 2
        return lo

    def gather(c):
        b = c % NBUF
        return pltpu.async_copy(
            tab_hbm.at[idx_v.at[pl.ds(c * CHUNK, CHUNK)]], bufs[b], gsems[b]
        )

    c0.wait()
    gathers = [gather(0), gather(1)]
    c1.wait()
    img_v[pl.ds(NIMG, LANES)] = jnp.full((LANES,), SEQ + 1, jnp.int32)
    p = count_lt(base)
    writes = [None] * NCHUNK

    for c in range(NCHUNK):
        b = c % NBUF
        buf = bufs[b]
        cb = base + c * CHUNK

        gathers[c].wait()

        nvec = D // LANES

        hi = count_lt(base + (c + 1) * CHUNK)

        @pl.loop(p, hi, init_carry=(jnp.int32(0), jnp.int32(0)))
        def ov_state(pp, carry):
            cnt, rmask = carry
            r = img_at(pp) - cb
            is_last = img_at(pp + 1) != img_at(pp)

            @pl.when(is_last)
            def _():
                pltpu.async_copy(vis_hbm.at[pp], buf.at[r], osem)

            return (cnt + jnp.where(is_last, 1, 0),
                    rmask | lax.shift_left(jnp.int32(1), r))

        n_issued, row_mask = ov_state

        def row_body(r, carry):
            skip = (lax.shift_right_logical(row_mask, r) & 1) == 1

            @pl.when(jnp.logical_not(skip))
            def _():
                @plsc.parallel_loop(0, nvec, unroll=16)
                def _(m):
                    sl = pl.ds(m * LANES, LANES)
                    buf[r, sl] = buf[r, sl] * SCALE

            return carry

        lax.fori_loop(0, CHUNK, row_body, jnp.int32(0))

        @pl.loop(0, n_issued)
        def _(_k):
            pltpu.make_async_copy(vis_hbm.at[0], buf.at[0], osem).wait()

        p = hi

        writes[c] = pltpu.async_copy(buf, out_hbm.at[pl.ds(cb, CHUNK)], wsems[b])

        if c + 2 < NCHUNK:
            if c >= 1:
                writes[c - 1].wait()
            gathers.append(gather(c + 2))

    writes[NCHUNK - 3].wait()
    writes[NCHUNK - 2].wait()
    writes[NCHUNK - 1].wait()


def kernel(input_ids, image_indices, vision_hidden_states, embed_table):
    mesh = plsc.VectorSubcoreMesh(core_axis_name="c", subcore_axis_name="s")
    f = pl.kernel(
        _sc_body,
        out_type=jax.ShapeDtypeStruct((SEQ, D), jnp.float32),
        mesh=mesh,
        scratch_types=[
            pltpu.VMEM((RPW,), jnp.int32),
            pltpu.VMEM((NIMG + LANES,), jnp.int32),
            pltpu.VMEM((CHUNK, D), jnp.float32),
            pltpu.VMEM((CHUNK, D), jnp.float32),
            pltpu.VMEM((CHUNK, D), jnp.float32),
            pltpu.SemaphoreType.DMA,
            pltpu.SemaphoreType.DMA,
            pltpu.SemaphoreType.DMA,
            pltpu.SemaphoreType.DMA,
            pltpu.SemaphoreType.DMA,
            pltpu.SemaphoreType.DMA,
            pltpu.SemaphoreType.DMA,
        ],
    )
    return f(input_ids, image_indices, vision_hidden_states, embed_table)
